# Initial kernel scaffold; baseline (speedup 1.0000x reference)
import functools

import jax
import jax.numpy as jnp
from jax import lax
from jax.experimental import pallas as pl
from jax.experimental.pallas import tpu as pltpu
from jax.experimental.pallas import tpu_sc as plsc

_N = 16384
_K = 50
_D = 64
_MARGIN = 1.0
_LAMBDA_W = 1.0

_NC = 2
_NS = 16
_NW = _NC * _NS
_NPW = _N // _NW
_BLK = 8
_NBLK = _NPW // _BLK
_QCH = 128
_NQCH = _NPW // _QCH


def _dot_slices(ref, off, q0, q1, q2, q3):
    t0 = ref[pl.ds(off + 0, 16)]
    t1 = ref[pl.ds(off + 16, 16)]
    t2 = ref[pl.ds(off + 32, 16)]
    t3 = ref[pl.ds(off + 48, 16)]
    return jnp.sum(q0 * t0 + q1 * t1 + q2 * t2 + q3 * t3)


@functools.partial(
    pl.kernel,
    out_type=jax.ShapeDtypeStruct((_NW, 16), jnp.float32),
    mesh=plsc.VectorSubcoreMesh(
        core_axis_name="c", subcore_axis_name="s",
        num_cores=_NC, num_subcores=_NS),
    scratch_types=[
        pltpu.VMEM((_D,), jnp.float32),
        pltpu.VMEM((_QCH,), jnp.int32),
        pltpu.VMEM((_QCH, _D), jnp.float32),
        pltpu.VMEM((_NPW * _D,), jnp.float32),
        pltpu.VMEM((_BLK,), jnp.int32),
        pltpu.VMEM((_BLK, _K), jnp.int32),
        pltpu.VMEM((_BLK * _D,), jnp.float32),
        pltpu.VMEM((_BLK, _K * _D), jnp.float32),
        pltpu.VMEM((16,), jnp.float32),
        pltpu.SemaphoreType.DMA,
    ],
)
def _sc_loss(h_x, referents, pos_s, neg_s, attr_i, etab, atab, rel,
             out, relv, idxc, arows, qbuf, posidx, negidx, posrows,
             negrows, outv, sem):
    wid = lax.axis_index("s") * _NC + lax.axis_index("c")
    base = wid * _NPW

    pltpu.sync_copy(rel, relv)
    r0 = relv[pl.ds(0, 16)]
    r1 = relv[pl.ds(16, 16)]
    r2 = relv[pl.ds(32, 16)]
    r3 = relv[pl.ds(48, 16)]

    qbuf2d = qbuf.reshape(_NPW, _D)
    for c in range(_NQCH):
        cb = base + c * _QCH
        pltpu.sync_copy(referents.at[pl.ds(cb, _QCH)], idxc)
        pltpu.async_copy(
            h_x.at[idxc], qbuf2d.at[pl.ds(c * _QCH, _QCH)], sem).wait()
        pltpu.sync_copy(attr_i.at[pl.ds(cb, _QCH)], idxc)
        pltpu.async_copy(atab.at[idxc], arows, sem).wait()
        arflat = arows.reshape(_QCH * _D)

        def qbody(i, _, c=c, arflat=arflat):
            qo = (c * _QCH + i) * _D
            ao = i * _D
            h0 = qbuf[pl.ds(qo + 0, 16)] + arflat[pl.ds(ao + 0, 16)]
            h1 = qbuf[pl.ds(qo + 16, 16)] + arflat[pl.ds(ao + 16, 16)]
            h2 = qbuf[pl.ds(qo + 32, 16)] + arflat[pl.ds(ao + 32, 16)]
            h3 = qbuf[pl.ds(qo + 48, 16)] + arflat[pl.ds(ao + 48, 16)]
            qbuf[pl.ds(qo + 0, 16)] = h0 * r0 - h2 * r2
            qbuf[pl.ds(qo + 16, 16)] = h1 * r1 - h3 * r3
            qbuf[pl.ds(qo + 32, 16)] = h0 * r2 + h2 * r0
            qbuf[pl.ds(qo + 48, 16)] = h1 * r3 + h3 * r1
            return 0

        lax.fori_loop(0, _QCH, qbody, 0)

    posrows2d = posrows.reshape(_BLK, _D)

    def block_body(b, loss):
        n0 = base + b * _BLK
        pltpu.sync_copy(pos_s.at[pl.ds(n0, _BLK)], posidx)
        pltpu.sync_copy(neg_s.at[pl.ds(n0, _BLK)], negidx)
        cps = [pltpu.async_copy(etab.at[posidx], posrows2d, sem)]
        for j in range(_BLK):
            cps.append(pltpu.async_copy(
                etab.at[negidx.at[j]], negrows.at[j].reshape(_K, _D), sem))
        for cp in cps:
            cp.wait()

        for j in range(_BLK):
            qo = (b * _BLK + j) * _D
            q0 = qbuf[pl.ds(qo + 0, 16)]
            q1 = qbuf[pl.ds(qo + 16, 16)]
            q2 = qbuf[pl.ds(qo + 32, 16)]
            q3 = qbuf[pl.ds(qo + 48, 16)]
            s_pos = _dot_slices(posrows, j * _D, q0, q1, q2, q3)
            m = _MARGIN - s_pos
            nrow = negrows.at[j]

            def kbody(kk, ls, nrow=nrow, q0=q0, q1=q1, q2=q2, q3=q3, m=m):
                s = _dot_slices(nrow, kk * _D, q0, q1, q2, q3)
                return ls + jnp.maximum(m + s, 0.0)

            loss = lax.fori_loop(0, _K, kbody, loss)
        return loss

    loss = lax.fori_loop(0, _NBLK, block_body, jnp.float32(0.0))
    loss = loss * (_LAMBDA_W / (_N * _K))

    outv[...] = jnp.where(
        lax.broadcasted_iota(jnp.int32, (16,), 0) == 0, loss, 0.0)
    pltpu.sync_copy(outv, out.at[wid])


def kernel(h_x, referents, positive_samples, negative_samples, ref_attribs,
           entity_table, attrib_table, rel):
    out = _sc_loss(
        h_x,
        referents.astype(jnp.int32),
        positive_samples.astype(jnp.int32),
        negative_samples.astype(jnp.int32),
        ref_attribs.astype(jnp.int32),
        entity_table,
        attrib_table,
        rel,
    )
    loss = jnp.sum(out)
    return (loss, h_x)


# speedup vs baseline: 2.3034x; 2.3034x over previous
import functools

import jax
import jax.numpy as jnp
from jax import lax
from jax.experimental import pallas as pl
from jax.experimental.pallas import tpu as pltpu
from jax.experimental.pallas import tpu_sc as plsc

_N = 16384
_K = 50
_D = 64
_MARGIN = 1.0
_LAMBDA_W = 1.0

_NC = 2
_NS = 16
_NW = _NC * _NS
_NPW = _N // _NW
_BLK = 8
_NBLK = _NPW // _BLK
_QCH = 128
_NQCH = _NPW // _QCH


def _row_vregs(ref, row):
    return (ref[row, pl.ds(0, 16)], ref[row, pl.ds(16, 16)],
            ref[row, pl.ds(32, 16)], ref[row, pl.ds(48, 16)])


def _acc_row(ref, row, q0, q1, q2, q3):
    t0, t1, t2, t3 = _row_vregs(ref, row)
    return q0 * t0 + q1 * t1 + q2 * t2 + q3 * t3


def _transpose_sum(ttile, iota16):
    base = iota16 * 16
    s = plsc.load_gather(ttile, [base])
    for l in range(1, 16):
        s = s + plsc.load_gather(ttile, [base + l])
    return s


def _sc_loss(h_x, referents, pos_s, neg_s, attr_i, etab, atab, rel,
             out, relv, idxc, arows, qbuf, posidx, negidx, posrows,
             negrows, ttile, mbuf, outv, sem):
    wid = lax.axis_index("s") * _NC + lax.axis_index("c")
    base = wid * _NPW

    pltpu.sync_copy(rel, relv)
    r0 = relv[pl.ds(0, 16)]
    r1 = relv[pl.ds(16, 16)]
    r2 = relv[pl.ds(32, 16)]
    r3 = relv[pl.ds(48, 16)]

    for c in range(_NQCH):
        cb = base + c * _QCH
        pltpu.sync_copy(referents.at[pl.ds(cb, _QCH)], idxc)
        pltpu.async_copy(
            h_x.at[idxc], qbuf.at[pl.ds(c * _QCH, _QCH)], sem).wait()
        pltpu.sync_copy(attr_i.at[pl.ds(cb, _QCH)], idxc)
        pltpu.async_copy(atab.at[idxc], arows, sem).wait()

        def qbody(i, _, c=c):
            row = c * _QCH + i
            a0, a1, a2, a3 = _row_vregs(arows, i)
            h0 = qbuf[row, pl.ds(0, 16)] + a0
            h1 = qbuf[row, pl.ds(16, 16)] + a1
            h2 = qbuf[row, pl.ds(32, 16)] + a2
            h3 = qbuf[row, pl.ds(48, 16)] + a3
            qbuf[row, pl.ds(0, 16)] = h0 * r0 - h2 * r2
            qbuf[row, pl.ds(16, 16)] = h1 * r1 - h3 * r3
            qbuf[row, pl.ds(32, 16)] = h0 * r2 + h2 * r0
            qbuf[row, pl.ds(48, 16)] = h1 * r3 + h3 * r1
            return 0

        lax.fori_loop(0, _QCH, qbody, 0)

    iota16 = lax.broadcasted_iota(jnp.int32, (16,), 0)
    ngrp = _K // 16
    ktail = _K - ngrp * 16
    tailmask = iota16 < ktail

    def block_body(b, loss_vec):
        n0 = base + b * _BLK
        pltpu.sync_copy(pos_s.at[pl.ds(n0, _BLK)], posidx)
        pltpu.sync_copy(neg_s.at[pl.ds(n0, _BLK)], negidx)
        cps = [pltpu.async_copy(etab.at[posidx], posrows, sem)]
        for j in range(_BLK):
            cps.append(pltpu.async_copy(
                etab.at[negidx.at[j]], negrows.at[j], sem))
        for cp in cps:
            cp.wait()

        for j in range(_BLK):
            q0, q1, q2, q3 = _row_vregs(qbuf, b * _BLK + j)
            ttile[pl.ds(j * 16, 16)] = _acc_row(posrows, j, q0, q1, q2, q3)
        mbuf[...] = _MARGIN - _transpose_sum(ttile, iota16)

        def jbody(j, lv):
            q0, q1, q2, q3 = _row_vregs(qbuf, b * _BLK + j)
            m = plsc.load_gather(mbuf, [jnp.full((16,), j, jnp.int32)])
            for g in range(ngrp):
                for lane in range(16):
                    ttile[pl.ds(lane * 16, 16)] = _acc_row(
                        negrows.at[j], g * 16 + lane, q0, q1, q2, q3)
                s = _transpose_sum(ttile, iota16)
                lv = lv + jnp.maximum(m + s, 0.0)
            for lane in range(ktail):
                ttile[pl.ds(lane * 16, 16)] = _acc_row(
                    negrows.at[j], ngrp * 16 + lane, q0, q1, q2, q3)
            s = _transpose_sum(ttile, iota16)
            lv = lv + jnp.where(tailmask, jnp.maximum(m + s, 0.0), 0.0)
            return lv

        return lax.fori_loop(0, _BLK, jbody, loss_vec)

    loss_vec = lax.fori_loop(
        0, _NBLK, block_body, jnp.zeros((16,), jnp.float32))
    outv[...] = loss_vec * (_LAMBDA_W / (_N * _K))
    pltpu.sync_copy(outv, out.at[wid])


@functools.cache
def _build():
    return pl.kernel(
        _sc_loss,
        out_type=jax.ShapeDtypeStruct((_NW, 16), jnp.float32),
        mesh=plsc.VectorSubcoreMesh(
            core_axis_name="c", subcore_axis_name="s",
            num_cores=_NC, num_subcores=_NS),
        compiler_params=pltpu.CompilerParams(needs_layout_passes=False, use_tc_tiling_on_sc=False),
        scratch_types=[
            pltpu.VMEM((_D,), jnp.float32),
            pltpu.VMEM((_QCH,), jnp.int32),
            pltpu.VMEM((_QCH, _D), jnp.float32),
            pltpu.VMEM((_NPW, _D), jnp.float32),
            pltpu.VMEM((_BLK,), jnp.int32),
            pltpu.VMEM((_BLK, _K), jnp.int32),
            pltpu.VMEM((_BLK, _D), jnp.float32),
            pltpu.VMEM((_BLK, _K, _D), jnp.float32),
            pltpu.VMEM((256,), jnp.float32),
            pltpu.VMEM((16,), jnp.float32),
            pltpu.VMEM((16,), jnp.float32),
            pltpu.SemaphoreType.DMA,
        ],
    )


def kernel(h_x, referents, positive_samples, negative_samples, ref_attribs,
           entity_table, attrib_table, rel):
    out = _build()(
        h_x,
        referents.astype(jnp.int32),
        positive_samples.astype(jnp.int32),
        negative_samples.astype(jnp.int32),
        ref_attribs.astype(jnp.int32),
        entity_table,
        attrib_table,
        rel,
    )
    loss = jnp.sum(out)
    return (loss, h_x)


# speedup vs baseline: 2.5123x; 1.0907x over previous
import functools

import jax
import jax.numpy as jnp
from jax import lax
from jax.experimental import pallas as pl
from jax.experimental.pallas import tpu as pltpu
from jax.experimental.pallas import tpu_sc as plsc

_N = 16384
_K = 50
_D = 64
_MARGIN = 1.0
_LAMBDA_W = 1.0

_NC = 2
_NS = 16
_NW = _NC * _NS
_NPW = _N // _NW
_BLK = 8
_NBLK = _NPW // _BLK
_QCH = 128
_NQCH = _NPW // _QCH


def _row_vregs(ref, row):
    return (ref[row, pl.ds(0, 16)], ref[row, pl.ds(16, 16)],
            ref[row, pl.ds(32, 16)], ref[row, pl.ds(48, 16)])


def _acc_row(ref, row, q0, q1, q2, q3):
    t0, t1, t2, t3 = _row_vregs(ref, row)
    return q0 * t0 + q1 * t1 + q2 * t2 + q3 * t3


def _transpose_sum(ttile, iota16):
    base = iota16 * 16
    s = plsc.load_gather(ttile, [base])
    for l in range(1, 16):
        s = s + plsc.load_gather(ttile, [base + l])
    return s


def _sc_loss(h_x, referents, pos_s, neg_s, attr_i, etab, atab, rel,
             out, relv, idxc, arows, qbuf, posidx, negidx, posrows,
             negrows, ttile, mbuf, outv, sem, sem2):
    wid = lax.axis_index("s") * _NC + lax.axis_index("c")
    base = wid * _NPW

    pltpu.sync_copy(rel, relv)
    r0 = relv[pl.ds(0, 16)]
    r1 = relv[pl.ds(16, 16)]
    r2 = relv[pl.ds(32, 16)]
    r3 = relv[pl.ds(48, 16)]

    for c in range(_NQCH):
        cb = base + c * _QCH
        pltpu.sync_copy(referents.at[pl.ds(cb, _QCH)], idxc)
        pltpu.async_copy(
            h_x.at[idxc], qbuf.at[pl.ds(c * _QCH, _QCH)], sem).wait()
        pltpu.sync_copy(attr_i.at[pl.ds(cb, _QCH)], idxc)
        pltpu.async_copy(atab.at[idxc], arows, sem).wait()

        def qbody(i, _, c=c):
            row = c * _QCH + i
            a0, a1, a2, a3 = _row_vregs(arows, i)
            h0 = qbuf[row, pl.ds(0, 16)] + a0
            h1 = qbuf[row, pl.ds(16, 16)] + a1
            h2 = qbuf[row, pl.ds(32, 16)] + a2
            h3 = qbuf[row, pl.ds(48, 16)] + a3
            qbuf[row, pl.ds(0, 16)] = h0 * r0 - h2 * r2
            qbuf[row, pl.ds(16, 16)] = h1 * r1 - h3 * r3
            qbuf[row, pl.ds(32, 16)] = h0 * r2 + h2 * r0
            qbuf[row, pl.ds(48, 16)] = h1 * r3 + h3 * r1
            return 0

        lax.fori_loop(0, _QCH, qbody, 0)

    iota16 = lax.broadcasted_iota(jnp.int32, (16,), 0)
    ngrp = _K // 16
    ktail = _K - ngrp * 16
    tailmask = iota16 < ktail
    sems = (sem, sem2)

    def issue(b, s):
        n0 = base + b * _BLK
        pltpu.sync_copy(pos_s.at[pl.ds(n0, _BLK)], posidx.at[s])
        pltpu.sync_copy(neg_s.at[pl.ds(n0, _BLK)], negidx.at[s])
        pltpu.async_copy(etab.at[posidx.at[s]], posrows.at[s], sems[s])
        for j in range(_BLK):
            pltpu.async_copy(
                etab.at[negidx.at[s].at[j]], negrows.at[s].at[j], sems[s])

    def drain(s):
        pltpu.make_async_copy(
            etab.at[posidx.at[s]], posrows.at[s], sems[s]).wait()
        for j in range(_BLK):
            pltpu.make_async_copy(
                etab.at[negidx.at[s].at[j]], negrows.at[s].at[j],
                sems[s]).wait()

    def compute(b, s, loss_vec):
        for j in range(_BLK):
            q0, q1, q2, q3 = _row_vregs(qbuf, b * _BLK + j)
            ttile[pl.ds(j * 16, 16)] = _acc_row(
                posrows.at[s], j, q0, q1, q2, q3)
        mbuf[...] = _MARGIN - _transpose_sum(ttile, iota16)

        def jbody(j, lv):
            q0, q1, q2, q3 = _row_vregs(qbuf, b * _BLK + j)
            m = plsc.load_gather(mbuf, [jnp.full((16,), j, jnp.int32)])
            nrow = negrows.at[s].at[j]
            for g in range(ngrp):
                for lane in range(16):
                    ttile[pl.ds(lane * 16, 16)] = _acc_row(
                        nrow, g * 16 + lane, q0, q1, q2, q3)
                sv = _transpose_sum(ttile, iota16)
                lv = lv + jnp.maximum(m + sv, 0.0)
            for lane in range(ktail):
                ttile[pl.ds(lane * 16, 16)] = _acc_row(
                    nrow, ngrp * 16 + lane, q0, q1, q2, q3)
            sv = _transpose_sum(ttile, iota16)
            lv = lv + jnp.where(tailmask, jnp.maximum(m + sv, 0.0), 0.0)
            return lv

        return lax.fori_loop(0, _BLK, jbody, loss_vec)

    issue(0, 0)
    issue(1, 1)

    def pipe_body(i, loss_vec):
        b = i * 2
        drain(0)
        loss_vec = compute(b, 0, loss_vec)
        issue(b + 2, 0)
        drain(1)
        loss_vec = compute(b + 1, 1, loss_vec)
        issue(b + 3, 1)
        return loss_vec

    loss_vec = lax.fori_loop(
        0, _NBLK // 2 - 1, pipe_body, jnp.zeros((16,), jnp.float32))
    drain(0)
    loss_vec = compute(_NBLK - 2, 0, loss_vec)
    drain(1)
    loss_vec = compute(_NBLK - 1, 1, loss_vec)

    outv[...] = loss_vec * (_LAMBDA_W / (_N * _K))
    pltpu.sync_copy(outv, out.at[wid])


@functools.cache
def _build():
    return pl.kernel(
        _sc_loss,
        out_type=jax.ShapeDtypeStruct((_NW, 16), jnp.float32),
        mesh=plsc.VectorSubcoreMesh(
            core_axis_name="c", subcore_axis_name="s",
            num_cores=_NC, num_subcores=_NS),
        compiler_params=pltpu.CompilerParams(needs_layout_passes=False, use_tc_tiling_on_sc=False),
        scratch_types=[
            pltpu.VMEM((_D,), jnp.float32),
            pltpu.VMEM((_QCH,), jnp.int32),
            pltpu.VMEM((_QCH, _D), jnp.float32),
            pltpu.VMEM((_NPW, _D), jnp.float32),
            pltpu.VMEM((2, _BLK), jnp.int32),
            pltpu.VMEM((2, _BLK, _K), jnp.int32),
            pltpu.VMEM((2, _BLK, _D), jnp.float32),
            pltpu.VMEM((2, _BLK, _K, _D), jnp.float32),
            pltpu.VMEM((256,), jnp.float32),
            pltpu.VMEM((16,), jnp.float32),
            pltpu.VMEM((16,), jnp.float32),
            pltpu.SemaphoreType.DMA,
            pltpu.SemaphoreType.DMA,
        ],
    )


def kernel(h_x, referents, positive_samples, negative_samples, ref_attribs,
           entity_table, attrib_table, rel):
    out = _build()(
        h_x,
        referents.astype(jnp.int32),
        positive_samples.astype(jnp.int32),
        negative_samples.astype(jnp.int32),
        ref_attribs.astype(jnp.int32),
        entity_table,
        attrib_table,
        rel,
    )
    loss = jnp.sum(out)
    return (loss, h_x)
